# Initial kernel scaffold; baseline (speedup 1.0000x reference)
import jax
import jax.numpy as jnp
from jax.experimental import pallas as pl


def kernel(x, edge_index, W1l, b1l, W1r, W2l, b2l, W2r, Wh, bh):
    raise NotImplementedError("write your pallas kernel here")


# speedup vs baseline: 4.5370x; 4.5370x over previous
import functools

import jax
import jax.numpy as jnp
from jax import lax
from jax.experimental import pallas as pl
from jax.experimental.pallas import tpu as pltpu
from jax.experimental.pallas import tpu_sc as plsc

NC = 2
NS = 16
NW = NC * NS


def _pad_rows(N):
    return -(-N // (8 * NS)) * (8 * NS)


def _make_sc_aggregate(N, E, D, C):
    EPT = E // NW
    NCHUNK = EPT // C
    NP = _pad_rows(N)
    RPT = NP // NS
    mesh = plsc.VectorSubcoreMesh(core_axis_name="c", subcore_axis_name="s")

    @functools.partial(
        pl.kernel, mesh=mesh,
        out_type=jax.ShapeDtypeStruct((NC, NP, D), jnp.float32),
        scratch_types=[
            pltpu.VMEM((C,), jnp.int32),
            pltpu.VMEM((C,), jnp.int32),
            pltpu.VMEM((C, D), jnp.float32),
            pltpu.VMEM_SHARED((NP, D), jnp.float32),
            pltpu.SemaphoreType.DMA,
        ],
    )
    def agg(feat_hbm, src_hbm, dst_hbm, z_hbm, out_hbm,
            src_v, dst_v, rows_v, acc_sh, sem):
        c = lax.axis_index("c")
        s = lax.axis_index("s")
        wid = c * NS + s

        pltpu.sync_copy(z_hbm.at[pl.ds(s * RPT, RPT)],
                        acc_sh.at[pl.ds(s * RPT, RPT)])
        plsc.subcore_barrier()

        ebase = wid * EPT

        def body(i, carry):
            b = ebase + i * C
            pltpu.sync_copy(src_hbm.at[pl.ds(b, C)], src_v)
            pltpu.sync_copy(dst_hbm.at[pl.ds(b, C)], dst_v)
            pltpu.async_copy(feat_hbm.at[src_v], rows_v, sem).wait()
            pltpu.sync_copy(rows_v, acc_sh.at[dst_v], add=True)
            return carry

        lax.fori_loop(0, NCHUNK, body, 0)
        plsc.subcore_barrier()

        pltpu.sync_copy(acc_sh.at[pl.ds(s * RPT, RPT)],
                        out_hbm.at[c].at[pl.ds(s * RPT, RPT)])

    return agg


def _make_sc_counts(N, E, C):
    EPT = E // NW
    NCHUNK = EPT // C
    NP = _pad_rows(N)
    RPT = NP // NS
    mesh = plsc.VectorSubcoreMesh(core_axis_name="c", subcore_axis_name="s")

    @functools.partial(
        pl.kernel, mesh=mesh,
        out_type=jax.ShapeDtypeStruct((NC, NP, 128), jnp.float32),
        scratch_types=[
            pltpu.VMEM((C,), jnp.int32),
            pltpu.VMEM((C, 128), jnp.float32),
            pltpu.VMEM_SHARED((NP, 128), jnp.float32),
        ],
    )
    def cntk(ones_hbm, dst_hbm, z_hbm, out_hbm, dst_v, ones_v, acc_sh):
        c = lax.axis_index("c")
        s = lax.axis_index("s")
        wid = c * NS + s

        pltpu.sync_copy(z_hbm.at[pl.ds(s * RPT, RPT)],
                        acc_sh.at[pl.ds(s * RPT, RPT)])
        pltpu.sync_copy(ones_hbm, ones_v)
        plsc.subcore_barrier()

        ebase = wid * EPT

        def body(i, carry):
            b = ebase + i * C
            pltpu.sync_copy(dst_hbm.at[pl.ds(b, C)], dst_v)
            pltpu.sync_copy(ones_v, acc_sh.at[dst_v], add=True)
            return carry

        lax.fori_loop(0, NCHUNK, body, 0)
        plsc.subcore_barrier()

        pltpu.sync_copy(acc_sh.at[pl.ds(s * RPT, RPT)],
                        out_hbm.at[c].at[pl.ds(s * RPT, RPT)])

    return cntk


_DN = (((1,), (1,)), ((), ()))


def _dot_t(a, w):
    return lax.dot_general(a, w, _DN, preferred_element_type=jnp.float32,
                           precision=lax.Precision.HIGHEST)


def _tc1_body(s1_ref, cnt_ref, x_ref, wl_ref, bl_ref, wr_ref, o_ref):
    a = s1_ref[0] + s1_ref[1]
    cnt = cnt_ref[0] + cnt_ref[1]
    mean = a / jnp.maximum(cnt, 1.0)
    t = _dot_t(mean, wl_ref[...]) + _dot_t(x_ref[...], wr_ref[...])
    o_ref[...] = jnp.maximum(t + bl_ref[...], 0.0)


def _tc2_body(s2_ref, cnt_ref, h_ref, w2l_ref, b2l_ref, w2r_ref,
              wh_ref, bh_ref, o_ref):
    a = s2_ref[0] + s2_ref[1]
    cnt = cnt_ref[0] + cnt_ref[1]
    mean = a / jnp.maximum(cnt, 1.0)
    t = _dot_t(mean, w2l_ref[...]) + _dot_t(h_ref[...], w2r_ref[...])
    h2 = jnp.maximum(t + b2l_ref[...], 0.0)
    o_ref[...] = _dot_t(h2, wh_ref[...]) + bh_ref[...]


def _full(shape):
    return pl.BlockSpec(shape, lambda i: (0,) * len(shape))


def kernel(x, edge_index, W1l, b1l, W1r, W2l, b2l, W2r, Wh, bh):
    N, D_IN = x.shape
    E = edge_index.shape[1]
    D_OUT = Wh.shape[0]
    C = 80

    src = edge_index[0]
    dst = edge_index[1]
    NP = _pad_rows(N)
    zeros = jnp.zeros((NP, D_IN), jnp.float32)
    ones_rows = jnp.ones((C, 128), jnp.float32)

    c1 = _make_sc_counts(N, E, C)(ones_rows, dst, zeros)
    s1 = _make_sc_aggregate(N, E, D_IN, C)(x, src, dst, zeros)

    cnt_n = c1[:, :N, :1]

    bN = 1000
    grid = (N // bN,)
    h = pl.pallas_call(
        _tc1_body,
        grid=grid,
        in_specs=[
            pl.BlockSpec((NC, bN, D_IN), lambda i: (0, i, 0)),
            pl.BlockSpec((NC, bN, 1), lambda i: (0, i, 0)),
            pl.BlockSpec((bN, D_IN), lambda i: (i, 0)),
            _full((D_IN, D_IN)),
            _full((1, D_IN)),
            _full((D_IN, D_IN)),
        ],
        out_specs=pl.BlockSpec((bN, D_IN), lambda i: (i, 0)),
        out_shape=jax.ShapeDtypeStruct((N, D_IN), jnp.float32),
    )(s1, cnt_n, x, W1l, b1l.reshape(1, -1), W1r)

    s2 = _make_sc_aggregate(N, E, D_IN, C)(h, src, dst, zeros)

    out = pl.pallas_call(
        _tc2_body,
        grid=grid,
        in_specs=[
            pl.BlockSpec((NC, bN, D_IN), lambda i: (0, i, 0)),
            pl.BlockSpec((NC, bN, 1), lambda i: (0, i, 0)),
            pl.BlockSpec((bN, D_IN), lambda i: (i, 0)),
            _full((D_IN, D_IN)),
            _full((1, D_IN)),
            _full((D_IN, D_IN)),
            _full((D_OUT, D_IN)),
            _full((1, D_OUT)),
        ],
        out_specs=pl.BlockSpec((bN, D_OUT), lambda i: (i, 0)),
        out_shape=jax.ShapeDtypeStruct((N, D_OUT), jnp.float32),
    )(s2, cnt_n, h, W2l, b2l.reshape(1, -1), W2r, Wh, bh.reshape(1, -1))

    return out


# speedup vs baseline: 10.5319x; 2.3213x over previous
import functools

import jax
import jax.numpy as jnp
from jax import lax
from jax.experimental import pallas as pl
from jax.experimental.pallas import tpu as pltpu
from jax.experimental.pallas import tpu_sc as plsc

NC = 2
NS = 16
NW = NC * NS


def _pad_rows(N):
    return -(-N // (8 * NS)) * (8 * NS)


def _make_sc_aggregate(N, E, D, C):
    EPT = E // NW
    NCHUNK = EPT // C
    NP = _pad_rows(N)
    RPT = NP // NS
    mesh = plsc.VectorSubcoreMesh(core_axis_name="c", subcore_axis_name="s")

    NB_ = 4
    assert NCHUNK % NB_ == 1
    NGRP = (NCHUNK - 1) // NB_

    @functools.partial(
        pl.kernel, mesh=mesh,
        out_type=jax.ShapeDtypeStruct((NC, NP, D), jnp.float32),
        scratch_types=(
            [pltpu.VMEM((C,), jnp.int32)] * NB_
            + [pltpu.VMEM((C,), jnp.int32)] * NB_
            + [pltpu.VMEM((C, D), jnp.float32)] * NB_
            + [pltpu.VMEM_SHARED((NP, D), jnp.float32)]
            + [pltpu.SemaphoreType.DMA] * (2 * NB_)
        ),
    )
    def agg(feat_hbm, src_hbm, dst_hbm, z_hbm, out_hbm, *scr):
        srcb = scr[0:NB_]
        dstb = scr[NB_:2 * NB_]
        rbuf = scr[2 * NB_:3 * NB_]
        acc_sh = scr[3 * NB_]
        si = scr[3 * NB_ + 1:3 * NB_ + 1 + NB_]
        sg = scr[3 * NB_ + 1 + NB_:]
        c = lax.axis_index("c")
        s = lax.axis_index("s")
        wid = c * NS + s
        ebase = wid * EPT

        def start_idx(ch, k):
            pltpu.async_copy(src_hbm.at[pl.ds(ebase + ch * C, C)],
                             srcb[k], si[k])
            pltpu.async_copy(dst_hbm.at[pl.ds(ebase + ch * C, C)],
                             dstb[k], si[k])

        def wait_idx(k):
            pltpu.make_async_copy(src_hbm.at[pl.ds(0, C)],
                                  srcb[k], si[k]).wait()
            pltpu.make_async_copy(dst_hbm.at[pl.ds(0, C)],
                                  dstb[k], si[k]).wait()

        def start_gather(k):
            pltpu.async_copy(feat_hbm.at[srcb[k]], rbuf[k], sg[k])

        def wait_gather(k):
            pltpu.make_async_copy(feat_hbm.at[srcb[k]],
                                  rbuf[k], sg[k]).wait()

        for k in range(NB_):
            start_idx(k, k)
        pltpu.sync_copy(z_hbm.at[pl.ds(s * RPT, RPT)],
                        acc_sh.at[pl.ds(s * RPT, RPT)])
        for k in range(NB_ - 1):
            wait_idx(k)
            start_gather(k)
        plsc.subcore_barrier()

        def body(p, carry):
            for b in range(NB_):
                ch = NB_ * p + b
                k = b
                j = (b + NB_ - 1) % NB_
                wait_gather(k)
                pltpu.sync_copy(rbuf[k], acc_sh.at[dstb[k]], add=True)

                @pl.when(ch + NB_ < NCHUNK)
                def _():
                    start_idx(ch + NB_, k)

                @pl.when(ch + NB_ - 1 < NCHUNK)
                def _():
                    wait_idx(j)
                    start_gather(j)
            return carry

        lax.fori_loop(0, NGRP, body, 0)
        wait_gather(0)
        pltpu.sync_copy(rbuf[0], acc_sh.at[dstb[0]], add=True)
        plsc.subcore_barrier()

        pltpu.sync_copy(acc_sh.at[pl.ds(s * RPT, RPT)],
                        out_hbm.at[c].at[pl.ds(s * RPT, RPT)])

    return agg


def _make_sc_counts(N, E, C):
    EPT = E // NW
    NCHUNK = EPT // C
    NP = _pad_rows(N)
    RPT = NP // NS
    mesh = plsc.VectorSubcoreMesh(core_axis_name="c", subcore_axis_name="s")

    assert NCHUNK % 2 == 1
    NPAIR = (NCHUNK - 1) // 2

    @functools.partial(
        pl.kernel, mesh=mesh,
        out_type=jax.ShapeDtypeStruct((NC, NP, 128), jnp.float32),
        scratch_types=[
            pltpu.VMEM((C,), jnp.int32), pltpu.VMEM((C,), jnp.int32),
            pltpu.VMEM((C, 128), jnp.float32),
            pltpu.VMEM_SHARED((NP, 128), jnp.float32),
            pltpu.SemaphoreType.DMA, pltpu.SemaphoreType.DMA,
        ],
    )
    def cntk(ones_hbm, dst_hbm, z_hbm, out_hbm,
             dst0, dst1, ones_v, acc_sh, si0, si1):
        c = lax.axis_index("c")
        s = lax.axis_index("s")
        wid = c * NS + s
        dstb, si = (dst0, dst1), (si0, si1)
        ebase = wid * EPT

        def start_idx(ch, k):
            pltpu.async_copy(dst_hbm.at[pl.ds(ebase + ch * C, C)],
                             dstb[k], si[k])

        def wait_idx(k):
            pltpu.make_async_copy(dst_hbm.at[pl.ds(0, C)],
                                  dstb[k], si[k]).wait()

        start_idx(0, 0)
        start_idx(1, 1)
        pltpu.sync_copy(z_hbm.at[pl.ds(s * RPT, RPT)],
                        acc_sh.at[pl.ds(s * RPT, RPT)])
        pltpu.sync_copy(ones_hbm, ones_v)
        wait_idx(0)
        plsc.subcore_barrier()

        def body(p, carry):
            for b in range(2):
                ch = 2 * p + b
                k, nk = b, 1 - b
                pltpu.sync_copy(ones_v, acc_sh.at[dstb[k]], add=True)

                @pl.when(ch < NCHUNK - 2)
                def _():
                    start_idx(ch + 2, k)
                wait_idx(nk)
            return carry

        lax.fori_loop(0, NPAIR, body, 0)
        pltpu.sync_copy(ones_v, acc_sh.at[dstb[0]], add=True)
        plsc.subcore_barrier()

        pltpu.sync_copy(acc_sh.at[pl.ds(s * RPT, RPT)],
                        out_hbm.at[c].at[pl.ds(s * RPT, RPT)])

    return cntk


_DN = (((1,), (1,)), ((), ()))


def _dot_t(a, w):
    return lax.dot_general(a, w, _DN, preferred_element_type=jnp.float32,
                           precision=lax.Precision.HIGHEST)


def _tc1_body(s1_ref, cnt_ref, x_ref, wl_ref, bl_ref, wr_ref, o_ref):
    a = s1_ref[0] + s1_ref[1]
    cnt = cnt_ref[0] + cnt_ref[1]
    mean = a / jnp.maximum(cnt, 1.0)
    t = _dot_t(mean, wl_ref[...]) + _dot_t(x_ref[...], wr_ref[...])
    o_ref[...] = jnp.maximum(t + bl_ref[...], 0.0)


def _tc2_body(s2_ref, cnt_ref, h_ref, w2l_ref, b2l_ref, w2r_ref,
              wh_ref, bh_ref, o_ref):
    a = s2_ref[0] + s2_ref[1]
    cnt = cnt_ref[0] + cnt_ref[1]
    mean = a / jnp.maximum(cnt, 1.0)
    t = _dot_t(mean, w2l_ref[...]) + _dot_t(h_ref[...], w2r_ref[...])
    h2 = jnp.maximum(t + b2l_ref[...], 0.0)
    o_ref[...] = _dot_t(h2, wh_ref[...]) + bh_ref[...]


def _full(shape):
    return pl.BlockSpec(shape, lambda i: (0,) * len(shape))


def kernel(x, edge_index, W1l, b1l, W1r, W2l, b2l, W2r, Wh, bh):
    N, D_IN = x.shape
    E = edge_index.shape[1]
    D_OUT = Wh.shape[0]
    C = 80

    src = edge_index[0]
    dst = edge_index[1]
    NP = _pad_rows(N)
    zeros = jnp.zeros((NP, D_IN), jnp.float32)
    ones_rows = jnp.ones((C, 128), jnp.float32)

    c1 = _make_sc_counts(N, E, C)(ones_rows, dst, zeros)
    s1 = _make_sc_aggregate(N, E, D_IN, C)(x, src, dst, zeros)

    cnt_n = c1[:, :N, :1]

    bN = 1000
    grid = (N // bN,)
    h = pl.pallas_call(
        _tc1_body,
        grid=grid,
        in_specs=[
            pl.BlockSpec((NC, bN, D_IN), lambda i: (0, i, 0)),
            pl.BlockSpec((NC, bN, 1), lambda i: (0, i, 0)),
            pl.BlockSpec((bN, D_IN), lambda i: (i, 0)),
            _full((D_IN, D_IN)),
            _full((1, D_IN)),
            _full((D_IN, D_IN)),
        ],
        out_specs=pl.BlockSpec((bN, D_IN), lambda i: (i, 0)),
        out_shape=jax.ShapeDtypeStruct((N, D_IN), jnp.float32),
    )(s1, cnt_n, x, W1l, b1l.reshape(1, -1), W1r)

    s2 = _make_sc_aggregate(N, E, D_IN, C)(h, src, dst, zeros)

    out = pl.pallas_call(
        _tc2_body,
        grid=grid,
        in_specs=[
            pl.BlockSpec((NC, bN, D_IN), lambda i: (0, i, 0)),
            pl.BlockSpec((NC, bN, 1), lambda i: (0, i, 0)),
            pl.BlockSpec((bN, D_IN), lambda i: (i, 0)),
            _full((D_IN, D_IN)),
            _full((1, D_IN)),
            _full((D_IN, D_IN)),
            _full((D_OUT, D_IN)),
            _full((1, D_OUT)),
        ],
        out_specs=pl.BlockSpec((bN, D_OUT), lambda i: (i, 0)),
        out_shape=jax.ShapeDtypeStruct((N, D_OUT), jnp.float32),
    )(s2, cnt_n, h, W2l, b2l.reshape(1, -1), W2r, Wh, bh.reshape(1, -1))

    return out
